# baseline (speedup 1.0000x reference)
import functools

import jax
import jax.numpy as jnp
from jax import lax
from jax.experimental import pallas as pl
from jax.experimental.pallas import tpu as pltpu
from jax.experimental.pallas import tpu_sc as plsc

_KB = 1024
_LANES = 128


def _topk_body(feat_ref, w_ref, aux_ref, items_ref,
               idx_ref,
               h_s, pmax_s, pj_s, *, n_b, n_k, nsteps, kb):
    k = pl.program_id(0)
    kbv = kb // _LANES

    @pl.when(k == 0)
    def _init():
        b = aux_ref[0:1, 0:16]
        gamma = aux_ref[1:2, 0:16]
        beta = aux_ref[2:3, 0:16]
        h = lax.dot_general(feat_ref[...], w_ref[...],
                            (((1,), (1,)), ((), ())),
                            preferred_element_type=jnp.float32) + b
        mu = jnp.mean(h, axis=0, keepdims=True)
        var = jnp.mean((h - mu) ** 2, axis=0, keepdims=True)
        h = (h - mu) / jnp.sqrt(var + 1e-5)
        h = gamma * h + beta
        h = jnp.where(h >= 0, h, 0.01 * h)
        h_s[...] = h
        pmax_s[...] = jnp.full((n_b, _LANES), -jnp.inf, jnp.float32)
        pj_s[...] = jnp.zeros((n_b, _LANES), jnp.int32)

    score = lax.dot_general(h_s[...], items_ref[...],
                            (((1,), (1,)), ((), ())),
                            preferred_element_type=jnp.float32)

    def _scan(s):
        pmax = pmax_s[...]
        pj = pj_s[...]
        for j in range(kbv):
            v = s[:, j * _LANES:(j + 1) * _LANES]
            c = v > pmax
            pmax = jnp.maximum(pmax, v)
            pj = jnp.where(c, k * kbv + j, pj)
        pmax_s[...] = pmax
        pj_s[...] = pj

    @pl.when(k < nsteps - 1)
    def _main():
        _scan(score)

    @pl.when(k == nsteps - 1)
    def _tail():
        colmask = lax.broadcasted_iota(jnp.int32, (1, kb), 1) < (n_k - k * kb)
        _scan(jnp.where(colmask, score, -jnp.inf))
        pmax = pmax_s[...]
        pj = pj_s[...]
        m = jnp.max(pmax, axis=1, keepdims=True)
        lane = lax.broadcasted_iota(jnp.int32, (n_b, _LANES), 1)
        cand = pj * _LANES + lane
        big = jnp.int32(2 ** 30)
        idx_ref[...] = jnp.min(jnp.where(pmax == m, cand, big),
                               axis=1, keepdims=True)


def _top1(item_feature, all_items, W, aux):
    n_b = item_feature.shape[0]
    n_k, d = all_items.shape
    nsteps = pl.cdiv(n_k, _KB)
    body = functools.partial(_topk_body, n_b=n_b, n_k=n_k,
                             nsteps=nsteps, kb=_KB)
    idx = pl.pallas_call(
        body,
        grid=(nsteps,),
        in_specs=[
            pl.BlockSpec((n_b, item_feature.shape[1]), lambda k: (0, 0)),
            pl.BlockSpec(W.shape, lambda k: (0, 0)),
            pl.BlockSpec(aux.shape, lambda k: (0, 0)),
            pl.BlockSpec((_KB, d), lambda k: (k, 0)),
        ],
        out_specs=pl.BlockSpec((n_b, 1), lambda k: (0, 0)),
        out_shape=jax.ShapeDtypeStruct((n_b, 1), jnp.int32),
        scratch_shapes=[
            pltpu.VMEM((n_b, d), jnp.float32),
            pltpu.VMEM((n_b, _LANES), jnp.float32),
            pltpu.VMEM((n_b, _LANES), jnp.int32),
        ],
    )(item_feature, W, aux, all_items)
    return idx


def _sc_gather(table, idx):
    info = plsc.get_sparse_core_info()
    nw = info.num_cores * info.num_subcores
    n = idx.shape[0]
    d = table.shape[1]
    b_per_w = n // nw
    mesh = plsc.VectorSubcoreMesh(core_axis_name="c", subcore_axis_name="s")

    @functools.partial(
        pl.kernel, mesh=mesh,
        compiler_params=pltpu.CompilerParams(use_tc_tiling_on_sc=False),
        out_type=jax.ShapeDtypeStruct((n, d), jnp.float32),
        scratch_types=[
            pltpu.VMEM((b_per_w,), jnp.int32),
            pltpu.VMEM((b_per_w, d), jnp.float32),
            pltpu.SemaphoreType.DMA,
        ],
    )
    def gather(table_hbm, idx_hbm, out_hbm, idx_v, rows_v, sem):
        wid = lax.axis_index("s") * info.num_cores + lax.axis_index("c")
        base = wid * b_per_w
        pltpu.sync_copy(idx_hbm.at[pl.ds(base, b_per_w)], idx_v)
        pltpu.async_copy(table_hbm.at[idx_v], rows_v, sem).wait()
        pltpu.sync_copy(rows_v, out_hbm.at[pl.ds(base, b_per_w)])

    return gather(table, idx)


def _cosine_body(orig_ref, sort_ref, loss_ref, mean_ref, *, n_b):
    eps = 1e-6
    o = orig_ref[...]
    s = sort_ref[...]
    dot = jnp.sum(o * s, axis=1, keepdims=True)
    na = jnp.sqrt(jnp.sum(o * o, axis=1, keepdims=True))
    nc = jnp.sqrt(jnp.sum(s * s, axis=1, keepdims=True))
    sim = dot / (jnp.maximum(na, eps) * jnp.maximum(nc, eps))
    sim = (sim + 1.0) * 0.5
    loss_ref[...] = jnp.mean((sim - 0.5) ** 2).reshape(1, 1)
    mean_ref[...] = jnp.mean(sim).reshape(1, 1)


def _cosine_loss(orig_feat, sort_feat):
    n_b, d = orig_feat.shape
    loss, mean = pl.pallas_call(
        functools.partial(_cosine_body, n_b=n_b),
        out_shape=[
            jax.ShapeDtypeStruct((1, 1), jnp.float32),
            jax.ShapeDtypeStruct((1, 1), jnp.float32),
        ],
    )(orig_feat, sort_feat)
    return loss.reshape(()), mean.reshape(())


def kernel(user_item_id, item_feature, all_items, W, b, gamma, beta):
    n_b = item_feature.shape[0]
    aux = jnp.zeros((8, W.shape[1]), jnp.float32)
    aux = aux.at[0, :16].set(b).at[1, :16].set(gamma).at[2, :16].set(beta)

    idx = _top1(item_feature, all_items, W, aux)
    sorted_items = idx.reshape(-1)

    cat_idx = jnp.concatenate([user_item_id[:, 1], sorted_items])
    feats = _sc_gather(all_items, cat_idx)
    orig_feat = feats[:n_b]
    sort_feat = feats[n_b:]
    similarity_loss, mean_sim = _cosine_loss(orig_feat, sort_feat)
    return (sorted_items, similarity_loss, mean_sim)
